# initial kernel scaffold (unmeasured)
import jax
import jax.numpy as jnp
from jax import lax
from jax.experimental import pallas as pl
from jax.experimental.pallas import tpu as pltpu


def kernel(
    x,
):
    def body(*refs):
        pass

    out_shape = jax.ShapeDtypeStruct(..., jnp.float32)
    return pl.pallas_call(body, out_shape=out_shape)(...)


# baseline (device time: 31593 ns/iter reference)
import jax
import jax.numpy as jnp
from jax import lax
from jax.experimental import pallas as pl
from jax.experimental.pallas import tpu as pltpu

N_Y = 2


def kernel(x):
    m, n = x.shape
    n_half = n // N_Y
    out_m = N_Y * m

    def body(x_ref, out_ref, send_buf, recv_buf, send_sem, recv_sem):
        my_x = lax.axis_index("x")
        my_y = lax.axis_index("y")
        my_z = lax.axis_index("z")
        peer_y = 1 - my_y

        barrier = pltpu.get_barrier_semaphore()
        pl.semaphore_signal(
            barrier, inc=1,
            device_id=(my_x, peer_y, my_z),
            device_id_type=pl.DeviceIdType.MESH,
        )
        pl.semaphore_wait(barrier, 1)

        @pl.when(my_y == 0)
        def _():
            send_buf[...] = x_ref[:, n_half:].astype(jnp.bfloat16)

        @pl.when(my_y == 1)
        def _():
            send_buf[...] = x_ref[:, :n_half].astype(jnp.bfloat16)

        rdma = pltpu.make_async_remote_copy(
            src_ref=send_buf,
            dst_ref=recv_buf,
            send_sem=send_sem,
            recv_sem=recv_sem,
            device_id=(my_x, peer_y, my_z),
            device_id_type=pl.DeviceIdType.MESH,
        )
        rdma.start()

        @pl.when(my_y == 0)
        def _():
            out_ref[:m, :] = x_ref[:, :n_half].astype(jnp.bfloat16)

        @pl.when(my_y == 1)
        def _():
            out_ref[m:, :] = x_ref[:, n_half:].astype(jnp.bfloat16)

        rdma.wait()

        @pl.when(my_y == 0)
        def _():
            out_ref[m:, :] = recv_buf[...]

        @pl.when(my_y == 1)
        def _():
            out_ref[:m, :] = recv_buf[...]

    return pl.pallas_call(
        body,
        out_shape=jax.ShapeDtypeStruct((out_m, n_half), jnp.bfloat16),
        in_specs=[pl.BlockSpec(memory_space=pltpu.VMEM)],
        out_specs=pl.BlockSpec(memory_space=pltpu.VMEM),
        scratch_shapes=[
            pltpu.VMEM((m, n_half), jnp.bfloat16),
            pltpu.VMEM((m, n_half), jnp.bfloat16),
            pltpu.SemaphoreType.DMA,
            pltpu.SemaphoreType.DMA,
        ],
        compiler_params=pltpu.CompilerParams(collective_id=0),
    )(x)


# device time: 31308 ns/iter; 1.0091x vs baseline; 1.0091x over previous
import jax
import jax.numpy as jnp
from jax import lax
from jax.experimental import pallas as pl
from jax.experimental.pallas import tpu as pltpu

N_Y = 2
N_CHUNKS = 4


def kernel(x):
    m, n = x.shape
    n_half = n // N_Y
    out_m = N_Y * m
    rows_per = m // N_CHUNKS

    def body(x_ref, out_ref, send_buf, send_sems, recv_sems):
        my_x = lax.axis_index("x")
        my_y = lax.axis_index("y")
        my_z = lax.axis_index("z")
        peer_y = 1 - my_y

        barrier = pltpu.get_barrier_semaphore()
        pl.semaphore_signal(
            barrier, inc=1,
            device_id=(my_x, peer_y, my_z),
            device_id_type=pl.DeviceIdType.MESH,
        )
        pl.semaphore_wait(barrier, 1)

        rdmas = []
        for c in range(N_CHUNKS):
            rows = pl.ds(c * rows_per, rows_per)

            @pl.when(my_y == 0)
            def _():
                send_buf[rows, :] = x_ref[rows, n_half:].astype(jnp.bfloat16)

            @pl.when(my_y == 1)
            def _():
                send_buf[rows, :] = x_ref[rows, :n_half].astype(jnp.bfloat16)

            rdma = pltpu.make_async_remote_copy(
                src_ref=send_buf.at[rows, :],
                dst_ref=out_ref.at[pl.ds(my_y * m + c * rows_per, rows_per), :],
                send_sem=send_sems.at[c],
                recv_sem=recv_sems.at[c],
                device_id=(my_x, peer_y, my_z),
                device_id_type=pl.DeviceIdType.MESH,
            )
            rdma.start()
            rdmas.append(rdma)

        @pl.when(my_y == 0)
        def _():
            out_ref[:m, :] = x_ref[:, :n_half].astype(jnp.bfloat16)

        @pl.when(my_y == 1)
        def _():
            out_ref[m:, :] = x_ref[:, n_half:].astype(jnp.bfloat16)

        for rdma in rdmas:
            rdma.wait_send()
        for rdma in rdmas:
            rdma.wait_recv()

    return pl.pallas_call(
        body,
        out_shape=jax.ShapeDtypeStruct((out_m, n_half), jnp.bfloat16),
        in_specs=[pl.BlockSpec(memory_space=pltpu.VMEM)],
        out_specs=pl.BlockSpec(memory_space=pltpu.VMEM),
        scratch_shapes=[
            pltpu.VMEM((m, n_half), jnp.bfloat16),
            pltpu.SemaphoreType.DMA((N_CHUNKS,)),
            pltpu.SemaphoreType.DMA((N_CHUNKS,)),
        ],
        compiler_params=pltpu.CompilerParams(collective_id=0),
    )(x)
